# initial kernel scaffold (unmeasured)
import jax
import jax.numpy as jnp
from jax import lax
from jax.experimental import pallas as pl
from jax.experimental.pallas import tpu as pltpu

Z = 4


def kernel(x, W):
    rows, _ = x.shape
    _, cols = W.shape
    n_total = Z * cols

    def body(x_ref, w_ref, out_ref, send_r, recv_r, send_l, recv_l):
        my_x = lax.axis_index("x")
        my_y = lax.axis_index("y")
        my_z = lax.axis_index("z")

        def desc(j, dz, send_sem, recv_sem):
            return pltpu.make_async_remote_copy(
                src_ref=out_ref.at[:, pl.ds(j * cols, cols)],
                dst_ref=out_ref.at[:, pl.ds(j * cols, cols)],
                send_sem=send_sem,
                recv_sem=recv_sem,
                device_id=(my_x, my_y, my_z + dz),
                device_id_type=pl.DeviceIdType.MESH,
            )

        out_ref[:, pl.ds(my_z * cols, cols)] = jnp.dot(
            x_ref[...], w_ref[...], preferred_element_type=jnp.float32
        )

        for h in range(Z - 1):
            if h > 0:
                @pl.when(my_z >= h)
                def _(h=h):
                    desc(my_z - h, -1, send_r.at[h - 1], recv_r.at[h - 1]).wait_recv()

            @pl.when((my_z >= h) & (my_z <= Z - 2))
            def _(h=h):
                desc(my_z - h, 1, send_r.at[h], recv_r.at[h]).start()

            if h > 0:
                @pl.when(my_z <= Z - 1 - h)
                def _(h=h):
                    desc(my_z + h, 1, send_l.at[h - 1], recv_l.at[h - 1]).wait_recv()

            @pl.when((my_z >= 1) & (my_z <= Z - 1 - h))
            def _(h=h):
                desc(my_z + h, -1, send_l.at[h], recv_l.at[h]).start()

        @pl.when(my_z >= Z - 1)
        def _():
            desc(my_z - (Z - 1), -1, send_r.at[Z - 2], recv_r.at[Z - 2]).wait_recv()

        @pl.when(my_z <= 0)
        def _():
            desc(my_z + (Z - 1), 1, send_l.at[Z - 2], recv_l.at[Z - 2]).wait_recv()

        for h in range(Z - 1):
            @pl.when((my_z >= h) & (my_z <= Z - 2))
            def _(h=h):
                desc(my_z - h, 1, send_r.at[h], recv_r.at[h]).wait_send()

            @pl.when((my_z >= 1) & (my_z <= Z - 1 - h))
            def _(h=h):
                desc(my_z + h, -1, send_l.at[h], recv_l.at[h]).wait_send()

        tile = 2048
        nt = n_total // tile
        m = jnp.full((rows, 1), -jnp.inf, dtype=jnp.float32)
        for t in range(nt):
            m = jnp.maximum(
                m,
                jnp.max(out_ref[:, t * tile:(t + 1) * tile], axis=1, keepdims=True),
            )
        s = jnp.zeros((rows, 1), dtype=jnp.float32)
        for t in range(nt):
            e = jnp.exp(out_ref[:, t * tile:(t + 1) * tile] - m)
            out_ref[:, t * tile:(t + 1) * tile] = e
            s = s + jnp.sum(e, axis=1, keepdims=True)
        r = 1.0 / s
        for t in range(nt):
            out_ref[:, t * tile:(t + 1) * tile] = (
                out_ref[:, t * tile:(t + 1) * tile] * r
            )

    return pl.pallas_call(
        body,
        out_shape=jax.ShapeDtypeStruct((rows, n_total), jnp.float32),
        in_specs=[
            pl.BlockSpec(memory_space=pltpu.VMEM),
            pl.BlockSpec(memory_space=pltpu.VMEM),
        ],
        out_specs=pl.BlockSpec(memory_space=pltpu.VMEM),
        scratch_shapes=[
            pltpu.SemaphoreType.DMA((Z - 1,)),
            pltpu.SemaphoreType.DMA((Z - 1,)),
            pltpu.SemaphoreType.DMA((Z - 1,)),
            pltpu.SemaphoreType.DMA((Z - 1,)),
        ],
    )(x, W)


# baseline (device time: 671112 ns/iter reference)
import jax
import jax.numpy as jnp
from jax import lax
from jax.experimental import pallas as pl
from jax.experimental.pallas import tpu as pltpu

Z = 4


def kernel(x, W):
    rows, _ = x.shape
    _, cols = W.shape
    n_total = Z * cols

    def body(x_ref, w_ref, out_ref, wbuf_ref, ebuf_ref, stats_ref,
             lsem, ssr, rsr, ssl, rsl, sdr, rdr, sdl, rdl):
        my_x = lax.axis_index("x")
        my_y = lax.axis_index("y")
        my_z = lax.axis_index("z")

        def line_allgather(mk, send_r, recv_r, send_l, recv_l):
            for h in range(Z - 1):
                if h > 0:
                    @pl.when(my_z >= h)
                    def _(h=h):
                        mk(my_z - h, -1,
                           send_r.at[h - 1], recv_r.at[h - 1]).wait_recv()

                @pl.when((my_z >= h) & (my_z <= Z - 2))
                def _(h=h):
                    mk(my_z - h, 1, send_r.at[h], recv_r.at[h]).start()

                if h > 0:
                    @pl.when(my_z <= Z - 1 - h)
                    def _(h=h):
                        mk(my_z + h, 1,
                           send_l.at[h - 1], recv_l.at[h - 1]).wait_recv()

                @pl.when((my_z >= 1) & (my_z <= Z - 1 - h))
                def _(h=h):
                    mk(my_z + h, -1, send_l.at[h], recv_l.at[h]).start()

            @pl.when(my_z >= Z - 1)
            def _():
                mk(my_z - (Z - 1), -1,
                   send_r.at[Z - 2], recv_r.at[Z - 2]).wait_recv()

            @pl.when(my_z <= 0)
            def _():
                mk(my_z + (Z - 1), 1,
                   send_l.at[Z - 2], recv_l.at[Z - 2]).wait_recv()

        def drain_sends(mk, send_r, recv_r, send_l, recv_l):
            for h in range(Z - 1):
                @pl.when((my_z >= h) & (my_z <= Z - 2))
                def _(h=h):
                    mk(my_z - h, 1, send_r.at[h], recv_r.at[h]).wait_send()

                @pl.when((my_z >= 1) & (my_z <= Z - 1 - h))
                def _(h=h):
                    mk(my_z + h, -1, send_l.at[h], recv_l.at[h]).wait_send()

        def stat_desc(j, dz, send_sem, recv_sem):
            return pltpu.make_async_remote_copy(
                src_ref=stats_ref.at[j],
                dst_ref=stats_ref.at[j],
                send_sem=send_sem,
                recv_sem=recv_sem,
                device_id=(my_x, my_y, my_z + dz),
                device_id_type=pl.DeviceIdType.MESH,
            )

        def data_desc(j, dz, send_sem, recv_sem):
            slot = out_ref.at[:, pl.ds(j * cols, cols)]
            return pltpu.make_async_remote_copy(
                src_ref=slot,
                dst_ref=slot,
                send_sem=send_sem,
                recv_sem=recv_sem,
                device_id=(my_x, my_y, my_z + dz),
                device_id_type=pl.DeviceIdType.MESH,
            )

        tile = 1024
        nt = cols // tile

        def gemm_tile(t, s):
            cp_w = pltpu.make_async_copy(
                w_ref.at[:, pl.ds(t * tile, tile)], wbuf_ref, lsem
            )
            cp_w.start()
            cp_w.wait()
            ebuf_ref[...] = jnp.exp(jnp.dot(
                x_ref[...], wbuf_ref[...],
                preferred_element_type=jnp.float32,
            ))
            cp_e = pltpu.make_async_copy(
                ebuf_ref,
                out_ref.at[:, pl.ds(my_z * cols + t * tile, tile)],
                lsem,
            )
            cp_e.start()
            cp_e.wait()
            return s + jnp.sum(ebuf_ref[...], axis=1)

        s = lax.fori_loop(
            0, nt, gemm_tile, jnp.zeros((rows,), dtype=jnp.float32)
        )
        stats_ref[my_z, :] = s

        line_allgather(stat_desc, ssr, rsr, ssl, rsl)

        r = (1.0 / jnp.sum(stats_ref[:, :], axis=0))[:, None]

        def norm_tile(t, carry):
            slot_tile = out_ref.at[:, pl.ds(my_z * cols + t * tile, tile)]
            cp_in = pltpu.make_async_copy(slot_tile, ebuf_ref, lsem)
            cp_in.start()
            cp_in.wait()
            ebuf_ref[...] = ebuf_ref[...] * r
            cp_out = pltpu.make_async_copy(ebuf_ref, slot_tile, lsem)
            cp_out.start()
            cp_out.wait()
            return carry

        lax.fori_loop(0, nt, norm_tile, 0)

        line_allgather(data_desc, sdr, rdr, sdl, rdl)

        drain_sends(stat_desc, ssr, rsr, ssl, rsl)
        drain_sends(data_desc, sdr, rdr, sdl, rdl)

    return pl.pallas_call(
        body,
        out_shape=jax.ShapeDtypeStruct((rows, n_total), jnp.float32),
        in_specs=[
            pl.BlockSpec(memory_space=pltpu.VMEM),
            pl.BlockSpec(memory_space=pl.ANY),
        ],
        out_specs=pl.BlockSpec(memory_space=pl.ANY),
        scratch_shapes=[
            pltpu.VMEM((1024, 1024), jnp.float32),
            pltpu.VMEM((rows, 1024), jnp.float32),
            pltpu.VMEM((Z, rows), jnp.float32),
            pltpu.SemaphoreType.DMA,
            pltpu.SemaphoreType.DMA((Z - 1,)),
            pltpu.SemaphoreType.DMA((Z - 1,)),
            pltpu.SemaphoreType.DMA((Z - 1,)),
            pltpu.SemaphoreType.DMA((Z - 1,)),
            pltpu.SemaphoreType.DMA((Z - 1,)),
            pltpu.SemaphoreType.DMA((Z - 1,)),
            pltpu.SemaphoreType.DMA((Z - 1,)),
            pltpu.SemaphoreType.DMA((Z - 1,)),
        ],
    )(x, W)


# device time: 650265 ns/iter; 1.0321x vs baseline; 1.0321x over previous
import jax
import jax.numpy as jnp
from jax import lax
from jax.experimental import pallas as pl
from jax.experimental.pallas import tpu as pltpu

Z = 4
TILE = 1024
SUB = 2048


def kernel(x, W):
    rows, _ = x.shape
    kdim, cols = W.shape
    n_total = Z * cols
    nt = cols // TILE
    nsub = cols // SUB

    def body(x_ref, w_ref, out_ref, g_ref, wbuf_ref, ebuf_ref, stats_ref,
             wsem, esem, lsem, ssr, rsr, ssl, rsl, sdr, rdr, sdl, rdl):
        my_x = lax.axis_index("x")
        my_y = lax.axis_index("y")
        my_z = lax.axis_index("z")

        def stat_desc(j, dz, send_sem, recv_sem):
            return pltpu.make_async_remote_copy(
                src_ref=stats_ref.at[j],
                dst_ref=stats_ref.at[j],
                send_sem=send_sem,
                recv_sem=recv_sem,
                device_id=(my_x, my_y, my_z + dz),
                device_id_type=pl.DeviceIdType.MESH,
            )

        def data_desc(j, u, dz, send_sem, recv_sem):
            sl = g_ref.at[:, pl.ds(j * cols + u * SUB, SUB)]
            return pltpu.make_async_remote_copy(
                src_ref=sl,
                dst_ref=sl,
                send_sem=send_sem,
                recv_sem=recv_sem,
                device_id=(my_x, my_y, my_z + dz),
                device_id_type=pl.DeviceIdType.MESH,
            )

        def wtile_cp(t):
            return pltpu.make_async_copy(
                w_ref.at[:, pl.ds(t * TILE, TILE)],
                wbuf_ref.at[t % 2],
                wsem.at[t % 2],
            )

        def etile_cp(t):
            return pltpu.make_async_copy(
                ebuf_ref.at[t % 2],
                g_ref.at[:, pl.ds(my_z * cols + t * TILE, TILE)],
                esem.at[t % 2],
            )

        def send_own_sub(u):
            @pl.when(my_z <= Z - 2)
            def _():
                data_desc(my_z, u, 1, sdr.at[u], rdr.at[u]).start()

            @pl.when(my_z >= 1)
            def _():
                data_desc(my_z, u, -1, sdl.at[u], rdl.at[u]).start()

        wtile_cp(0).start()
        s = jnp.zeros((rows,), dtype=jnp.float32)
        for t in range(nt):
            if t + 1 < nt:
                wtile_cp(t + 1).start()
            wtile_cp(t).wait()
            if t >= 2:
                etile_cp(t - 2).wait()
            e = jnp.exp(jnp.dot(
                x_ref[...], wbuf_ref[t % 2],
                preferred_element_type=jnp.float32,
            ))
            ebuf_ref[t % 2, :, :] = e
            s = s + jnp.sum(e, axis=1)
            etile_cp(t).start()
            if t % 2 == 1 and t >= 3:
                send_own_sub((t - 3) // 2)
        etile_cp(nt - 2).wait()
        etile_cp(nt - 1).wait()
        send_own_sub(nsub - 1)

        stats_ref[my_z, :] = s
        for h in range(Z - 1):
            if h > 0:
                @pl.when(my_z >= h)
                def _(h=h):
                    stat_desc(my_z - h, -1,
                              ssr.at[h - 1], rsr.at[h - 1]).wait_recv()

            @pl.when((my_z >= h) & (my_z <= Z - 2))
            def _(h=h):
                stat_desc(my_z - h, 1, ssr.at[h], rsr.at[h]).start()

            if h > 0:
                @pl.when(my_z <= Z - 1 - h)
                def _(h=h):
                    stat_desc(my_z + h, 1,
                              ssl.at[h - 1], rsl.at[h - 1]).wait_recv()

            @pl.when((my_z >= 1) & (my_z <= Z - 1 - h))
            def _(h=h):
                stat_desc(my_z + h, -1, ssl.at[h], rsl.at[h]).start()

        @pl.when(my_z >= Z - 1)
        def _():
            stat_desc(my_z - (Z - 1), -1,
                      ssr.at[Z - 2], rsr.at[Z - 2]).wait_recv()

        @pl.when(my_z <= 0)
        def _():
            stat_desc(my_z + (Z - 1), 1,
                      ssl.at[Z - 2], rsl.at[Z - 2]).wait_recv()

        r = (1.0 / jnp.sum(stats_ref[:, :], axis=0))[:, None]

        def scale_tile(off):
            cin = pltpu.make_async_copy(
                g_ref.at[:, pl.ds(off, TILE)], ebuf_ref.at[0], lsem
            )
            cin.start()
            cin.wait()
            ebuf_ref[0, :, :] = ebuf_ref[0, :, :] * r
            cout = pltpu.make_async_copy(
                ebuf_ref.at[0], out_ref.at[:, pl.ds(off, TILE)], lsem
            )
            cout.start()
            cout.wait()

        def scale_sub(j, u):
            for i in range(SUB // TILE):
                scale_tile(j * cols + u * SUB + i * TILE)

        for u in range(nsub):
            scale_sub(my_z, u)

        for h in range(1, Z - 1):
            for u in range(nsub):
                i_ev = (h - 1) * nsub + u
                i_sd = h * nsub + u

                @pl.when(my_z >= h)
                def _(h=h, u=u, i=i_ev):
                    data_desc(my_z - h, u, -1, sdr.at[i], rdr.at[i]).wait_recv()

                @pl.when((my_z >= h) & (my_z <= Z - 2))
                def _(h=h, u=u, i=i_sd):
                    data_desc(my_z - h, u, 1, sdr.at[i], rdr.at[i]).start()

                @pl.when(my_z >= h)
                def _(h=h, u=u):
                    scale_sub(my_z - h, u)

                @pl.when(my_z <= Z - 1 - h)
                def _(h=h, u=u, i=i_ev):
                    data_desc(my_z + h, u, 1, sdl.at[i], rdl.at[i]).wait_recv()

                @pl.when((my_z >= 1) & (my_z <= Z - 1 - h))
                def _(h=h, u=u, i=i_sd):
                    data_desc(my_z + h, u, -1, sdl.at[i], rdl.at[i]).start()

                @pl.when(my_z <= Z - 1 - h)
                def _(h=h, u=u):
                    scale_sub(my_z + h, u)

        for u in range(nsub):
            i_ev = (Z - 2) * nsub + u

            @pl.when(my_z >= Z - 1)
            def _(u=u, i=i_ev):
                data_desc(my_z - (Z - 1), u, -1,
                          sdr.at[i], rdr.at[i]).wait_recv()
                scale_sub(my_z - (Z - 1), u)

            @pl.when(my_z <= 0)
            def _(u=u, i=i_ev):
                data_desc(my_z + (Z - 1), u, 1,
                          sdl.at[i], rdl.at[i]).wait_recv()
                scale_sub(my_z + (Z - 1), u)

        for h in range(Z - 1):
            @pl.when((my_z >= h) & (my_z <= Z - 2))
            def _(h=h):
                stat_desc(my_z - h, 1, ssr.at[h], rsr.at[h]).wait_send()

            @pl.when((my_z >= 1) & (my_z <= Z - 1 - h))
            def _(h=h):
                stat_desc(my_z + h, -1, ssl.at[h], rsl.at[h]).wait_send()

            for u in range(nsub):
                i_sd = h * nsub + u

                @pl.when((my_z >= h) & (my_z <= Z - 2))
                def _(h=h, u=u, i=i_sd):
                    data_desc(my_z - h, u, 1, sdr.at[i], rdr.at[i]).wait_send()

                @pl.when((my_z >= 1) & (my_z <= Z - 1 - h))
                def _(h=h, u=u, i=i_sd):
                    data_desc(my_z + h, u, -1,
                              sdl.at[i], rdl.at[i]).wait_send()

    nds = (Z - 1) * (cols // SUB)
    out, _gather = pl.pallas_call(
        body,
        out_shape=[
            jax.ShapeDtypeStruct((rows, n_total), jnp.float32),
            jax.ShapeDtypeStruct((rows, n_total), jnp.float32),
        ],
        in_specs=[
            pl.BlockSpec(memory_space=pltpu.VMEM),
            pl.BlockSpec(memory_space=pl.ANY),
        ],
        out_specs=[
            pl.BlockSpec(memory_space=pl.ANY),
            pl.BlockSpec(memory_space=pl.ANY),
        ],
        scratch_shapes=[
            pltpu.VMEM((2, kdim, TILE), jnp.float32),
            pltpu.VMEM((2, rows, TILE), jnp.float32),
            pltpu.VMEM((Z, rows), jnp.float32),
            pltpu.SemaphoreType.DMA((2,)),
            pltpu.SemaphoreType.DMA((2,)),
            pltpu.SemaphoreType.DMA,
            pltpu.SemaphoreType.DMA((Z - 1,)),
            pltpu.SemaphoreType.DMA((Z - 1,)),
            pltpu.SemaphoreType.DMA((Z - 1,)),
            pltpu.SemaphoreType.DMA((Z - 1,)),
            pltpu.SemaphoreType.DMA((nds,)),
            pltpu.SemaphoreType.DMA((nds,)),
            pltpu.SemaphoreType.DMA((nds,)),
            pltpu.SemaphoreType.DMA((nds,)),
        ],
    )(x, W)
    return out
